# speedup vs baseline: 3.7118x; 3.7118x over previous
import functools

import jax
import jax.numpy as jnp
from jax.experimental import pallas as pl
from jax.experimental.pallas import tpu as pltpu

E = 64
D = 768
DFF = 2048
T = 2048
BT = 64
NB = T // BT + E


def _routing_body(x_ref, rw_ref, eid_ref, wt_ref):
    x = x_ref[...]
    rw = rw_ref[...]
    logits = jax.lax.dot_general(
        x, rw, (((1,), (1,)), ((), ())), preferred_element_type=jnp.float32
    )
    lmax = jnp.max(logits, axis=1, keepdims=True)
    sumexp = jnp.sum(jnp.exp(logits - lmax), axis=1, keepdims=True)
    iota = jax.lax.broadcasted_iota(jnp.int32, (T, E), 1)
    eid = jnp.min(jnp.where(logits == lmax, iota, E), axis=1, keepdims=True)
    eid_ref[...] = eid
    wt_ref[...] = 1.0 / sumexp


def _moe_body(blk_e, tok_ref, wblk_ref, x_ref, g_ref, u_ref, d_ref, out_ref):
    j = pl.program_id(0)
    idx = tok_ref[0, 0, :]
    w = wblk_ref[0, 0, :]
    x = x_ref[...]
    iota_bt = jax.lax.broadcasted_iota(jnp.int32, (BT, T), 1)
    gat = (iota_bt == idx[:, None]).astype(jnp.float32)
    iota_tb = jax.lax.broadcasted_iota(jnp.int32, (T, BT), 0)
    sca = (iota_tb == idx[None, :]).astype(jnp.float32)

    xb = jax.lax.dot_general(
        gat, x, (((1,), (0,)), ((), ())), preferred_element_type=jnp.float32
    )
    gw = g_ref[0]
    uw = u_ref[0]
    dw = d_ref[0]
    hg = jax.lax.dot_general(
        xb, gw, (((1,), (1,)), ((), ())), preferred_element_type=jnp.float32
    )
    hu = jax.lax.dot_general(
        xb, uw, (((1,), (1,)), ((), ())), preferred_element_type=jnp.float32
    )
    h = hg * jax.nn.sigmoid(hg) * hu
    y = jax.lax.dot_general(
        h, dw, (((1,), (1,)), ((), ())), preferred_element_type=jnp.float32
    )
    y = y * w[:, None]

    @pl.when(j == 0)
    def _():
        out_ref[...] = jnp.zeros_like(out_ref)

    out_ref[...] += jax.lax.dot_general(
        sca, y, (((1,), (0,)), ((), ())), preferred_element_type=jnp.float32
    )


@jax.jit
def kernel(hidden_states, router_W, gate_W, up_W, down_W):
    b, s, d = hidden_states.shape
    x = hidden_states.reshape(-1, d).astype(jnp.float32)

    eid2, wt2 = pl.pallas_call(
        _routing_body,
        out_shape=(
            jax.ShapeDtypeStruct((T, 1), jnp.int32),
            jax.ShapeDtypeStruct((T, 1), jnp.float32),
        ),
    )(x, router_W)
    eid = eid2[:, 0]
    wt = wt2[:, 0]

    perm = jnp.argsort(eid)
    counts = jnp.zeros((E,), jnp.int32).at[eid].add(1)
    offsets = jnp.concatenate(
        [jnp.zeros((1,), jnp.int32), jnp.cumsum(counts)[:-1]]
    )
    nblk = (counts + BT - 1) // BT
    cumblk = jnp.cumsum(nblk)
    total_blocks = cumblk[-1]
    jarr = jnp.arange(NB, dtype=jnp.int32)
    ej = jnp.searchsorted(cumblk, jarr, side="right").astype(jnp.int32)
    e_last = jnp.searchsorted(cumblk, total_blocks - 1, side="right").astype(
        jnp.int32
    )
    ej = jnp.where(jarr < total_blocks, ej, e_last)
    within = jarr - (cumblk[ej] - nblk[ej])
    start = offsets[ej] + within * BT
    cnt = jnp.clip(counts[ej] - within * BT, 0, BT)
    cnt = jnp.where(jarr < total_blocks, cnt, 0)
    g = start[:, None] + jnp.arange(BT, dtype=jnp.int32)[None, :]
    valid = jnp.arange(BT, dtype=jnp.int32)[None, :] < cnt[:, None]
    tok = jnp.where(valid, perm[jnp.clip(g, 0, T - 1)], 0).astype(jnp.int32)
    wblk = jnp.where(valid, wt[tok], 0.0).astype(jnp.float32)

    grid_spec = pltpu.PrefetchScalarGridSpec(
        num_scalar_prefetch=1,
        grid=(NB,),
        in_specs=[
            pl.BlockSpec((1, 1, BT), lambda j, be: (j, 0, 0)),
            pl.BlockSpec((1, 1, BT), lambda j, be: (j, 0, 0)),
            pl.BlockSpec((T, D), lambda j, be: (0, 0)),
            pl.BlockSpec((1, DFF, D), lambda j, be: (be[j], 0, 0)),
            pl.BlockSpec((1, DFF, D), lambda j, be: (be[j], 0, 0)),
            pl.BlockSpec((1, D, DFF), lambda j, be: (be[j], 0, 0)),
        ],
        out_specs=pl.BlockSpec((T, D), lambda j, be: (0, 0)),
    )
    out = pl.pallas_call(
        _moe_body,
        grid_spec=grid_spec,
        out_shape=jax.ShapeDtypeStruct((T, D), jnp.float32),
        compiler_params=pltpu.CompilerParams(
            dimension_semantics=("arbitrary",),
            vmem_limit_bytes=120 * 1024 * 1024,
        ),
    )(ej, tok.reshape(NB, 1, BT), wblk.reshape(NB, 1, BT), x, gate_W, up_W, down_W)
    return out.reshape(b, s, d)
